# baseline (speedup 1.0000x reference)
import functools

import jax
import jax.numpy as jnp
import numpy as np
from jax.experimental import pallas as pl

_G = 16
_N = 1000
_NH = 512
_F = 1024
_M = 40


def _proj_kernel(fa_ref, w_ref, b_ref, o_ref):
    o_ref[0] = jax.lax.dot_general(
        fa_ref[0], w_ref[...], (((1,), (1,)), ((), ())),
        preferred_element_type=jnp.float32) + b_ref[...]


def _main_kernel(pe_ref, iou_ref, q_ref, k_ref, p_ref, wgw_ref, wgb_ref,
                 cb_ref, y_ref):
    M = pe_ref.shape[1]
    pe = pe_ref[0]
    pe = jnp.concatenate(
        [pe, jnp.zeros((M, (_NH - 500) * 128), jnp.float32)], axis=1)
    pe2 = pe.reshape(M * _NH, 128)
    wgt = jax.lax.dot_general(
        wgw_ref[...], pe2, (((1,), (1,)), ((), ())),
        preferred_element_type=jnp.float32) + wgb_ref[...]
    lg = jnp.log(jnp.maximum(wgt, 1e-6)).reshape(2 * _G, M, _NH)

    iou = iou_ref[0]
    logc = jnp.log(jnp.asarray(1e-6, jnp.float32))
    liou = jnp.where(iou >= 1e-6, jnp.asarray(0.0, jnp.float32), logc)
    liou_e = liou[:, 0, :]
    liou_o = liou[:, 1, :]

    qblk = q_ref[0]
    k4 = k_ref[0]
    affs_e = []
    affs_o = []
    for j in range(_G):
        qj = qblk[:, j * 64:(j + 1) * 64]
        ae = jax.lax.dot_general(
            qj, k4[0, j * 64:(j + 1) * 64, :], (((1,), (0,)), ((), ())),
            preferred_element_type=jnp.float32) * 0.125 + liou_e
        ao = jax.lax.dot_general(
            qj, k4[1, j * 64:(j + 1) * 64, :], (((1,), (0,)), ((), ())),
            preferred_element_type=jnp.float32) * 0.125 + liou_o
        affs_e.append(ae[None])
        affs_o.append(ao[None])
    w_e = lg[:_G] + jnp.concatenate(affs_e, axis=0)
    w_o = lg[_G:] + jnp.concatenate(affs_o, axis=0)

    iota = jax.lax.broadcasted_iota(jnp.int32, (1, 1, _NH), 2)
    neg = -jnp.inf
    w_e = jnp.where(iota < 500, w_e, neg)
    w_o = jnp.where(iota < 500, w_o, neg)

    zero = jnp.asarray(0.0, jnp.float32)
    acc_e = jnp.zeros((_G, M, _NH), jnp.float32)
    acc_o = jnp.zeros((_G, M, _NH), jnp.float32)
    m0 = None
    z = None
    for t in range(10):
        mx = jnp.maximum(jnp.max(w_e, axis=2, keepdims=True),
                         jnp.max(w_o, axis=2, keepdims=True))
        if t == 0:
            m0 = mx
            e = jnp.ones_like(mx)
            z = e
        else:
            e = jnp.exp(mx - m0)
            z = z + e
        oh_e = w_e == mx
        oh_o = w_o == mx
        acc_e = acc_e + jnp.where(oh_e, e, zero)
        acc_o = acc_o + jnp.where(oh_o, e, zero)
        w_e = jnp.where(oh_e, neg, w_e)
        w_o = jnp.where(oh_o, neg, w_o)
    wsp_e = acc_e / z
    wsp_o = acc_o / z

    pp = p_ref[0]
    outs = []
    for j in range(_G):
        oe = jax.lax.dot_general(
            wsp_e[j], pp[0, :, j * 64:(j + 1) * 64], (((1,), (0,)), ((), ())),
            preferred_element_type=jnp.float32)
        oo = jax.lax.dot_general(
            wsp_o[j], pp[1, :, j * 64:(j + 1) * 64], (((1,), (0,)), ((), ())),
            preferred_element_type=jnp.float32)
        outs.append(oe + oo)
    y_ref[0] = jnp.concatenate(outs, axis=1) + cb_ref[...]


@jax.jit
def kernel(f_a, position_embedding, iou, WG_w, WG_b, WK_w, WK_b, WQ_w, WQ_b,
           conv_w, conv_b):
    N, C, F = f_a.shape
    fa = jnp.transpose(f_a, (1, 0, 2))
    conv2d = conv_w[:, :, 0, 0]
    wcat = jnp.concatenate([WQ_w, WK_w, conv2d], axis=0)
    bcat = jnp.concatenate(
        [WQ_b, WK_b, jnp.zeros_like(conv_b)])[None, :]

    qkp = pl.pallas_call(
        _proj_kernel,
        grid=(C, 3),
        in_specs=[
            pl.BlockSpec((1, N, F), lambda c, t: (c, 0, 0)),
            pl.BlockSpec((F, F), lambda c, t: (t, 0)),
            pl.BlockSpec((1, F), lambda c, t: (0, t)),
        ],
        out_specs=pl.BlockSpec((1, N, F), lambda c, t: (c, 0, t)),
        out_shape=jax.ShapeDtypeStruct((C, N, 3 * F), jnp.float32),
    )(fa, wcat, bcat)

    q = qkp[:, :, :F]
    kt = jnp.pad(qkp[:, :, F:2 * F], ((0, 0), (0, 2 * _NH - N), (0, 0)))
    kt = jnp.transpose(kt.reshape(C, _NH, 2, F), (0, 2, 3, 1))
    pp = jnp.pad(qkp[:, :, 2 * F:], ((0, 0), (0, 2 * _NH - N), (0, 0)))
    pp = jnp.transpose(pp.reshape(C, _NH, 2, F), (0, 2, 1, 3))
    ioup = jnp.pad(iou, ((0, 0), (0, 0), (0, 2 * _NH - N)))
    ioup = jnp.transpose(ioup.reshape(C, N, _NH, 2), (0, 1, 3, 2))
    pe4 = position_embedding.reshape(C, N, N * 64)
    wgw2 = jnp.concatenate([
        jnp.concatenate([WG_w, jnp.zeros_like(WG_w)], axis=1),
        jnp.concatenate([jnp.zeros_like(WG_w), WG_w], axis=1),
    ], axis=0)
    wgb2 = jnp.concatenate([WG_b, WG_b])[:, None]

    y = pl.pallas_call(
        _main_kernel,
        grid=(C, N // _M),
        in_specs=[
            pl.BlockSpec((1, _M, 64000), lambda c, i: (c, i, 0)),
            pl.BlockSpec((1, _M, 2, _NH), lambda c, i: (c, i, 0, 0)),
            pl.BlockSpec((1, _M, F), lambda c, i: (c, i, 0)),
            pl.BlockSpec((1, 2, F, _NH), lambda c, i: (c, 0, 0, 0)),
            pl.BlockSpec((1, 2, _NH, F), lambda c, i: (c, 0, 0, 0)),
            pl.BlockSpec((2 * _G, 128), lambda c, i: (0, 0)),
            pl.BlockSpec((2 * _G, 1), lambda c, i: (0, 0)),
            pl.BlockSpec((1, F), lambda c, i: (0, 0)),
        ],
        out_specs=pl.BlockSpec((1, _M, F), lambda c, i: (c, i, 0)),
        out_shape=jax.ShapeDtypeStruct((C, N, F), jnp.float32),
    )(pe4, ioup, q, kt, pp, wgw2, wgb2, conv_b[None, :])

    return jnp.transpose(y, (1, 0, 2))


# speedup vs baseline: 2.2301x; 2.2301x over previous
import functools

import jax
import jax.numpy as jnp
import numpy as np
from jax.experimental import pallas as pl
from jax.experimental.pallas import tpu as pltpu

_G = 16
_N = 1000
_NH = 512
_F = 1024
_M = 40


def _proj_kernel(fa_ref, w_ref, b_ref, o_ref):
    o_ref[0] = jax.lax.dot_general(
        fa_ref[0], w_ref[...], (((1,), (1,)), ((), ())),
        preferred_element_type=jnp.float32) + b_ref[...]


def _main_kernel(pe_ref, iou_ref, q_ref, k_ref, p_ref, wgw_ref, wgb_ref,
                 cb_ref, y_ref, wscr_ref):
    M = pe_ref.shape[1]
    s = pl.program_id(2)
    pe2 = pe_ref[0].reshape(M * _NH, 64)
    wgt = jax.lax.dot_general(
        wgw_ref[...], pe2, (((1,), (1,)), ((), ())),
        preferred_element_type=jnp.float32) + wgb_ref[...]
    lg = jnp.log(jnp.maximum(wgt, 1e-6)).reshape(_G, M, _NH)

    iou = iou_ref[0, 0]
    logc = jnp.log(jnp.asarray(1e-6, jnp.float32))
    liou = jnp.where(iou >= 1e-6, jnp.asarray(0.0, jnp.float32), logc)

    qblk = q_ref[0]
    k2 = k_ref[0, 0]
    affs = []
    for j in range(_G):
        qj = qblk[:, j * 64:(j + 1) * 64]
        a = jax.lax.dot_general(
            qj, k2[j * 64:(j + 1) * 64, :], (((1,), (0,)), ((), ())),
            preferred_element_type=jnp.float32) * 0.125 + liou
        affs.append(a[None])
    w = lg + jnp.concatenate(affs, axis=0)

    iota = jax.lax.broadcasted_iota(jnp.int32, (1, 1, _NH), 2)
    neg = -jnp.inf
    w = jnp.where(iota < _N - _NH * s, w, neg)

    @pl.when(s == 0)
    def _():
        wscr_ref[...] = w

    @pl.when(s == 1)
    def _():
        w0 = wscr_ref[...]
        w1 = w
        zero = jnp.asarray(0.0, jnp.float32)
        acc0 = jnp.zeros((_G, M, _NH), jnp.float32)
        acc1 = jnp.zeros((_G, M, _NH), jnp.float32)
        m0 = None
        z = None
        for t in range(10):
            mx = jnp.maximum(jnp.max(w0, axis=2, keepdims=True),
                             jnp.max(w1, axis=2, keepdims=True))
            if t == 0:
                m0 = mx
                e = jnp.ones_like(mx)
                z = e
            else:
                e = jnp.exp(mx - m0)
                z = z + e
            oh0 = w0 == mx
            oh1 = w1 == mx
            acc0 = acc0 + jnp.where(oh0, e, zero)
            acc1 = acc1 + jnp.where(oh1, e, zero)
            w0 = jnp.where(oh0, neg, w0)
            w1 = jnp.where(oh1, neg, w1)
        wsp0 = acc0 / z
        wsp1 = acc1 / z

        pp = p_ref[0]
        outs = []
        for j in range(_G):
            o0 = jax.lax.dot_general(
                wsp0[j], pp[0, :, j * 64:(j + 1) * 64],
                (((1,), (0,)), ((), ())), preferred_element_type=jnp.float32)
            o1 = jax.lax.dot_general(
                wsp1[j], pp[1, :, j * 64:(j + 1) * 64],
                (((1,), (0,)), ((), ())), preferred_element_type=jnp.float32)
            outs.append(o0 + o1)
        y_ref[0] = jnp.concatenate(outs, axis=1) + cb_ref[...]


@jax.jit
def kernel(f_a, position_embedding, iou, WG_w, WG_b, WK_w, WK_b, WQ_w, WQ_b,
           conv_w, conv_b):
    N, C, F = f_a.shape
    fa = jnp.transpose(f_a, (1, 0, 2))
    conv2d = conv_w[:, :, 0, 0]
    wcat = jnp.concatenate([WQ_w, WK_w, conv2d], axis=0)
    bcat = jnp.concatenate(
        [WQ_b, WK_b, jnp.zeros_like(conv_b)])[None, :]

    qkp = pl.pallas_call(
        _proj_kernel,
        grid=(C, 3),
        in_specs=[
            pl.BlockSpec((1, N, F), lambda c, t: (c, 0, 0)),
            pl.BlockSpec((F, F), lambda c, t: (t, 0)),
            pl.BlockSpec((1, F), lambda c, t: (0, t)),
        ],
        out_specs=pl.BlockSpec((1, N, F), lambda c, t: (c, 0, t)),
        out_shape=jax.ShapeDtypeStruct((C, N, 3 * F), jnp.float32),
    )(fa, wcat, bcat)

    q = qkp[:, :, :F]
    kt = jnp.pad(qkp[:, :, F:2 * F], ((0, 0), (0, 2 * _NH - N), (0, 0)))
    kt = jnp.transpose(kt.reshape(C, 2, _NH, F), (0, 1, 3, 2))
    pp = jnp.pad(qkp[:, :, 2 * F:], ((0, 0), (0, 2 * _NH - N), (0, 0)))
    pp = pp.reshape(C, 2, _NH, F)
    ioup = jnp.pad(iou, ((0, 0), (0, 0), (0, 2 * _NH - N)))
    ioup = jnp.transpose(ioup.reshape(C, N, 2, _NH), (0, 2, 1, 3))

    y = pl.pallas_call(
        _main_kernel,
        grid=(C, N // _M, 2),
        in_specs=[
            pl.BlockSpec((1, _M, _NH, 64), lambda c, i, s: (c, i, s, 0)),
            pl.BlockSpec((1, 1, _M, _NH), lambda c, i, s: (c, s, i, 0)),
            pl.BlockSpec((1, _M, F), lambda c, i, s: (c, i, 0)),
            pl.BlockSpec((1, 1, F, _NH), lambda c, i, s: (c, s, 0, 0)),
            pl.BlockSpec((1, 2, _NH, F), lambda c, i, s: (c, 0, 0, 0)),
            pl.BlockSpec((_G, 64), lambda c, i, s: (0, 0)),
            pl.BlockSpec((_G, 1), lambda c, i, s: (0, 0)),
            pl.BlockSpec((1, F), lambda c, i, s: (0, 0)),
        ],
        out_specs=pl.BlockSpec((1, _M, F), lambda c, i, s: (c, i, 0)),
        out_shape=jax.ShapeDtypeStruct((C, N, F), jnp.float32),
        scratch_shapes=[pltpu.VMEM((_G, _M, _NH), jnp.float32)],
    )(position_embedding, ioup, q, kt, pp, WG_w, WG_b[:, None],
      conv_b[None, :])

    return jnp.transpose(y, (1, 0, 2))
